# Initial kernel scaffold; baseline (speedup 1.0000x reference)
import jax
import jax.numpy as jnp
from jax.experimental import pallas as pl


def kernel(x):
    raise NotImplementedError("write your pallas kernel here")


# speedup vs baseline: 144.4588x; 144.4588x over previous
import jax
import jax.numpy as jnp
from jax import lax
from jax.experimental import pallas as pl
from jax.experimental.pallas import tpu as pltpu
from jax.experimental.pallas import tpu_sc as plsc

ROWS = 512
HALF = 256
COLS = 511
OUT_LEN = 131072
NB = 32
NQ = ROWS
CHUNK_Q = 64
N_CHUNKS = NQ // CHUNK_Q
BLK_W = 72


def _gather_col(blk, rows, col):
    cols = jnp.full((16,), col, dtype=jnp.int32)
    return plsc.load_gather(blk, [rows, cols])


def _body(x_hbm, xt_hbm, out_hbm, blk, out_blk, sem):
    b = lax.axis_index("s") * 2 + lax.axis_index("c")
    iota = lax.iota(jnp.int32, 16)

    for c in range(N_CHUNKS):
        if c == 0:
            pltpu.sync_copy(x_hbm.at[b, :, pl.ds(0, BLK_W)], blk)
            shift = 0
        elif c < N_CHUNKS - 1:
            pltpu.sync_copy(
                x_hbm.at[b, :, pl.ds(c * CHUNK_Q - 8, BLK_W)], blk
            )
            shift = 8
        else:
            pltpu.sync_copy(xt_hbm.at[b], blk)
            shift = 9

        if c == 0:
            for rg in range(16):
                rows_t = iota + (rg * 16)
                out_blk[0, pl.ds(rg * 16, 16)] = _gather_col(blk, rows_t, 0)

        lo = 1 if c == 0 else 0
        hi = CHUNK_Q - 1 if c == N_CHUNKS - 1 else CHUNK_Q

        def row_body(q, _):
            ct = q + shift
            for rg in range(16):
                rows_t = iota + (rg * 16)
                t = _gather_col(blk, rows_t, ct)
                bo = _gather_col(blk, rows_t + HALF, ct - 1)
                out_blk[q, pl.ds(rg * 16, 16)] = t + bo
            return _

        lax.fori_loop(lo, hi, row_body, None)

        if c == N_CHUNKS - 1:
            for rg in range(16):
                rows_b = iota + (rg * 16 + HALF)
                out_blk[CHUNK_Q - 1, pl.ds(rg * 16, 16)] = _gather_col(
                    blk, rows_b, BLK_W - 1
                )

        pltpu.sync_copy(out_blk, out_hbm.at[b, pl.ds(c * CHUNK_Q, CHUNK_Q), :])


@jax.jit
def kernel(x):
    xf = x.reshape(NB, ROWS, COLS)
    xt = xf[:, :, COLS - BLK_W:]
    mesh = plsc.VectorSubcoreMesh(core_axis_name="c", subcore_axis_name="s")
    out = pl.kernel(
        _body,
        out_type=jax.ShapeDtypeStruct((NB, NQ, HALF), jnp.float32),
        mesh=mesh,
        scratch_types=[
            pltpu.VMEM((ROWS, BLK_W), jnp.float32),
            pltpu.VMEM((CHUNK_Q, HALF), jnp.float32),
            pltpu.SemaphoreType.DMA,
        ],
        compiler_params=pltpu.CompilerParams(
            use_tc_tiling_on_sc=False, needs_layout_passes=False
        ),
    )(xf, xt)
    return out.reshape(*x.shape[:-2], OUT_LEN)
